# baseline (speedup 1.0000x reference)
import jax
import jax.numpy as jnp
from jax import lax
from jax.experimental import pallas as pl
from jax.experimental.pallas import tpu as pltpu
from jax.experimental.pallas import tpu_sc as plsc

B = 64
N = 20000
N_MAIN = 19968
N_TAIL = N - N_MAIN

SC_TILES = 84
N_SC = SC_TILES * 128

NUM_WORKERS = 32
OCTET = 8
QUARTER = N_SC // 4
CHUNK_N = 896
NUM_CHUNKS = QUARTER // CHUNK_N

TC_BB = 64
TC_BN = 1536
TC_JOFF = N_SC // TC_BN
TC_NBLK = (N_MAIN - N_SC) // TC_BN


def _sc_body(c_hbm, t_hbm, p_hbm, ct_hbm, tt_hbm, pt_hbm, out_hbm,
             c0, c1, t0, t1, p0, p1, ct_v, tt_v, pt_v, acc_v,
             sc0, sc1, st0, st1, sp0, sp1):
    nc = 2
    wid = lax.axis_index("s") * nc + lax.axis_index("c")
    o = wid // 4
    q = wid % 4
    b0 = pl.multiple_of(o * OCTET, 8)
    nbase = q * QUARTER
    zero = jnp.zeros((16,), jnp.float32)

    def start(k, cv, tv, pv, cs, ts, ps):
        n0 = pl.multiple_of(nbase + k * CHUNK_N, 128)
        pltpu.make_async_copy(
            c_hbm.at[pl.ds(b0, OCTET), pl.ds(n0, CHUNK_N)], cv, cs).start()
        pltpu.make_async_copy(
            t_hbm.at[pl.ds(b0, OCTET), :, pl.ds(n0, CHUNK_N)], tv, ts).start()
        pltpu.make_async_copy(
            p_hbm.at[pl.ds(b0, OCTET), :, pl.ds(n0, CHUNK_N)], pv, ps).start()

    def wait(k, cv, tv, pv, cs, ts, ps):
        n0 = pl.multiple_of(nbase + k * CHUNK_N, 128)
        pltpu.make_async_copy(
            c_hbm.at[pl.ds(b0, OCTET), pl.ds(n0, CHUNK_N)], cv, cs).wait()
        pltpu.make_async_copy(
            t_hbm.at[pl.ds(b0, OCTET), :, pl.ds(n0, CHUNK_N)], tv, ts).wait()
        pltpu.make_async_copy(
            p_hbm.at[pl.ds(b0, OCTET), :, pl.ds(n0, CHUNK_N)], pv, ps).wait()

    def compute(cv, tv, pv, carry):
        def group_body(g, gcarry):
            gsq, gwa = gcarry
            col = pl.ds(g * 16, 16)
            for bi in range(OCTET):
                c16 = cv[bi, col]
                w = jnp.where(c16 != 0, jnp.float32(1), jnp.float32(0))
                gwa = gwa + w
                d = pv[bi, 0, col] - tv[bi, 0, col]
                s = d * d
                for comp in range(1, 4):
                    d = pv[bi, comp, col] - tv[bi, comp, col]
                    s = s + d * d
                gsq = gsq + s * w
            return gsq, gwa

        return lax.fori_loop(0, CHUNK_N // 16, group_body, carry)

    start(0, c0, t0, p0, sc0, st0, sp0)

    def pair_body(i, carry):
        ka = 2 * i
        wait(ka, c0, t0, p0, sc0, st0, sp0)
        start(ka + 1, c1, t1, p1, sc1, st1, sp1)
        carry = compute(c0, t0, p0, carry)
        wait(ka + 1, c1, t1, p1, sc1, st1, sp1)
        start(ka + 2, c0, t0, p0, sc0, st0, sp0)
        return compute(c1, t1, p1, carry)

    carry = lax.fori_loop(0, NUM_CHUNKS // 2, pair_body, (zero, zero))
    wait(NUM_CHUNKS - 1, c0, t0, p0, sc0, st0, sp0)
    carry = compute(c0, t0, p0, carry)

    tb = wid * 2 * 4 * N_TAIL
    cb = wid * 2 * N_TAIL
    pltpu.sync_copy(ct_hbm.at[pl.ds(cb, 2 * N_TAIL)], ct_v)
    pltpu.sync_copy(tt_hbm.at[pl.ds(tb, 8 * N_TAIL)], tt_v)
    pltpu.sync_copy(pt_hbm.at[pl.ds(tb, 8 * N_TAIL)], pt_v)

    def tail_body(i, tcarry):
        sq, wa = tcarry
        bi = i // 2
        h = i % 2
        c16 = ct_v[pl.ds(bi * N_TAIL + h * 16, 16)]
        w = jnp.where(c16 != 0, jnp.float32(1), jnp.float32(0))
        wa = wa + w
        off = bi * 4 * N_TAIL + h * 16
        d = pt_v[pl.ds(off, 16)] - tt_v[pl.ds(off, 16)]
        s = d * d
        for comp in range(1, 4):
            off = bi * 4 * N_TAIL + comp * N_TAIL + h * 16
            d = pt_v[pl.ds(off, 16)] - tt_v[pl.ds(off, 16)]
            s = s + d * d
        return sq + s * w, wa

    sq, wa = lax.fori_loop(0, 4, tail_body, carry)

    acc_v[pl.ds(0, 16)] = sq
    acc_v[pl.ds(16, 16)] = wa
    pltpu.sync_copy(acc_v, out_hbm.at[wid])


def _sc_call(c2d, t3d, p3d, c_tail, t_tail, p_tail):
    mesh = plsc.VectorSubcoreMesh(core_axis_name="c", subcore_axis_name="s")
    return pl.kernel(
        _sc_body,
        mesh=mesh,
        out_type=jax.ShapeDtypeStruct((NUM_WORKERS, 32), jnp.float32),
        scratch_types=[
            pltpu.VMEM((OCTET, CHUNK_N), jnp.int32),
            pltpu.VMEM((OCTET, CHUNK_N), jnp.int32),
            pltpu.VMEM((OCTET, 4, CHUNK_N), jnp.float32),
            pltpu.VMEM((OCTET, 4, CHUNK_N), jnp.float32),
            pltpu.VMEM((OCTET, 4, CHUNK_N), jnp.float32),
            pltpu.VMEM((OCTET, 4, CHUNK_N), jnp.float32),
            pltpu.VMEM((2 * N_TAIL,), jnp.int32),
            pltpu.VMEM((8 * N_TAIL,), jnp.float32),
            pltpu.VMEM((8 * N_TAIL,), jnp.float32),
            pltpu.VMEM((32,), jnp.float32),
            pltpu.SemaphoreType.DMA,
            pltpu.SemaphoreType.DMA,
            pltpu.SemaphoreType.DMA,
            pltpu.SemaphoreType.DMA,
            pltpu.SemaphoreType.DMA,
            pltpu.SemaphoreType.DMA,
        ],
    )(c2d, t3d, p3d, c_tail, t_tail, p_tail)


def _tc_body(c_ref, t_ref, p_ref, num_ref, den_ref, accn, accd):
    i = pl.program_id(0)
    j = pl.program_id(1)

    @pl.when((i == 0) & (j == 0))
    def _():
        accn[...] = jnp.zeros((TC_BB, TC_BN), jnp.float32)
        accd[...] = jnp.zeros((TC_BB, TC_BN), jnp.float32)

    w = (c_ref[...] != 0).astype(jnp.float32)
    d = p_ref[...] - t_ref[...]
    s = jnp.sum(d * d, axis=1)
    accn[...] += s * w
    accd[...] += w

    @pl.when((i == B // TC_BB - 1) & (j == TC_NBLK - 1))
    def _():
        num_ref[0, 0] = jnp.sum(accn[...])
        den_ref[0, 0] = jnp.sum(accd[...])


def _tc_call(c2d, t3d, p3d):
    return pl.pallas_call(
        _tc_body,
        grid=(B // TC_BB, TC_NBLK),
        in_specs=[
            pl.BlockSpec((TC_BB, TC_BN), lambda i, j: (i, j + TC_JOFF)),
            pl.BlockSpec((TC_BB, 4, TC_BN), lambda i, j: (i, 0, j + TC_JOFF)),
            pl.BlockSpec((TC_BB, 4, TC_BN), lambda i, j: (i, 0, j + TC_JOFF)),
        ],
        out_specs=[
            pl.BlockSpec(memory_space=pltpu.SMEM),
            pl.BlockSpec(memory_space=pltpu.SMEM),
        ],
        out_shape=[
            jax.ShapeDtypeStruct((1, 1), jnp.float32),
            jax.ShapeDtypeStruct((1, 1), jnp.float32),
        ],
        scratch_shapes=[
            pltpu.VMEM((TC_BB, TC_BN), jnp.float32),
            pltpu.VMEM((TC_BB, TC_BN), jnp.float32),
        ],
    )(c2d, t3d, p3d)


@jax.jit
def _bbox_mse(c2d, t3d, p3d, c_tail, t_tail, p_tail):
    partials = _sc_call(c2d, t3d, p3d, c_tail, t_tail, p_tail)
    tc_num, tc_den = _tc_call(c2d, t3d, p3d)
    num = jnp.sum(partials[:, :16]) + tc_num[0, 0]
    den = (jnp.sum(partials[:, 16:]) + tc_den[0, 0]) * 4.0
    return num / den


def kernel(target_bbox, target_class, predicted_bbox):
    t3d = jnp.transpose(target_bbox, (0, 2, 1))
    p3d = jnp.transpose(predicted_bbox, (0, 2, 1))
    c2d = target_class.astype(jnp.int32)
    t_tail = t3d[:, :, N_MAIN:].reshape(-1)
    p_tail = p3d[:, :, N_MAIN:].reshape(-1)
    c_tail = c2d[:, N_MAIN:].reshape(-1)
    return _bbox_mse(c2d, t3d, p3d, c_tail, t_tail, p_tail)


# speedup vs baseline: 1.0837x; 1.0837x over previous
import jax
import jax.numpy as jnp
from jax import lax
from jax.experimental import pallas as pl
from jax.experimental.pallas import tpu as pltpu
from jax.experimental.pallas import tpu_sc as plsc

B = 64
N = 20000

SC_TILES = 84
N_SC = SC_TILES * 128

NUM_WORKERS = 32
OCTET = 8
QUARTER = N_SC // 4
CHUNK_N = 896
NUM_CHUNKS = QUARTER // CHUNK_N

TC_BB = 64
TC_BN = 1536
TC_JOFF = N_SC // TC_BN
TC_NBLK = -(-(N - N_SC) // TC_BN)


def _sc_body(c_hbm, t_hbm, p_hbm, out_hbm,
             c0, c1, t0, t1, p0, p1, acc_v,
             sc0, sc1, st0, st1, sp0, sp1):
    nc = 2
    wid = lax.axis_index("s") * nc + lax.axis_index("c")
    o = wid // 4
    q = wid % 4
    b0 = pl.multiple_of(o * OCTET, 8)
    nbase = q * QUARTER
    zero = jnp.zeros((16,), jnp.float32)

    def start(k, cv, tv, pv, cs, ts, ps):
        n0 = pl.multiple_of(nbase + k * CHUNK_N, 128)
        pltpu.make_async_copy(
            c_hbm.at[pl.ds(b0, OCTET), pl.ds(n0, CHUNK_N)], cv, cs).start()
        pltpu.make_async_copy(
            t_hbm.at[pl.ds(b0, OCTET), :, pl.ds(n0, CHUNK_N)], tv, ts).start()
        pltpu.make_async_copy(
            p_hbm.at[pl.ds(b0, OCTET), :, pl.ds(n0, CHUNK_N)], pv, ps).start()

    def wait(k, cv, tv, pv, cs, ts, ps):
        n0 = pl.multiple_of(nbase + k * CHUNK_N, 128)
        pltpu.make_async_copy(
            c_hbm.at[pl.ds(b0, OCTET), pl.ds(n0, CHUNK_N)], cv, cs).wait()
        pltpu.make_async_copy(
            t_hbm.at[pl.ds(b0, OCTET), :, pl.ds(n0, CHUNK_N)], tv, ts).wait()
        pltpu.make_async_copy(
            p_hbm.at[pl.ds(b0, OCTET), :, pl.ds(n0, CHUNK_N)], pv, ps).wait()

    def compute(cv, tv, pv, carry):
        def group_body(g, gcarry):
            gsq, gwa = gcarry
            col = pl.ds(g * 16, 16)
            for bi in range(OCTET):
                c16 = cv[bi, col]
                w = jnp.where(c16 != 0, jnp.float32(1), jnp.float32(0))
                gwa = gwa + w
                d = pv[bi, 0, col] - tv[bi, 0, col]
                s = d * d
                for comp in range(1, 4):
                    d = pv[bi, comp, col] - tv[bi, comp, col]
                    s = s + d * d
                gsq = gsq + s * w
            return gsq, gwa

        return lax.fori_loop(0, CHUNK_N // 16, group_body, carry)

    start(0, c0, t0, p0, sc0, st0, sp0)

    def pair_body(i, carry):
        ka = 2 * i
        wait(ka, c0, t0, p0, sc0, st0, sp0)
        start(ka + 1, c1, t1, p1, sc1, st1, sp1)
        carry = compute(c0, t0, p0, carry)
        wait(ka + 1, c1, t1, p1, sc1, st1, sp1)
        start(ka + 2, c0, t0, p0, sc0, st0, sp0)
        return compute(c1, t1, p1, carry)

    carry = lax.fori_loop(0, NUM_CHUNKS // 2, pair_body, (zero, zero))
    wait(NUM_CHUNKS - 1, c0, t0, p0, sc0, st0, sp0)
    sq, wa = compute(c0, t0, p0, carry)

    acc_v[pl.ds(0, 16)] = sq
    acc_v[pl.ds(16, 16)] = wa
    pltpu.sync_copy(acc_v, out_hbm.at[wid])


def _sc_call(c2d, t3d, p3d):
    mesh = plsc.VectorSubcoreMesh(core_axis_name="c", subcore_axis_name="s")
    return pl.kernel(
        _sc_body,
        mesh=mesh,
        out_type=jax.ShapeDtypeStruct((NUM_WORKERS, 32), jnp.float32),
        scratch_types=[
            pltpu.VMEM((OCTET, CHUNK_N), jnp.int32),
            pltpu.VMEM((OCTET, CHUNK_N), jnp.int32),
            pltpu.VMEM((OCTET, 4, CHUNK_N), jnp.float32),
            pltpu.VMEM((OCTET, 4, CHUNK_N), jnp.float32),
            pltpu.VMEM((OCTET, 4, CHUNK_N), jnp.float32),
            pltpu.VMEM((OCTET, 4, CHUNK_N), jnp.float32),
            pltpu.VMEM((32,), jnp.float32),
            pltpu.SemaphoreType.DMA,
            pltpu.SemaphoreType.DMA,
            pltpu.SemaphoreType.DMA,
            pltpu.SemaphoreType.DMA,
            pltpu.SemaphoreType.DMA,
            pltpu.SemaphoreType.DMA,
        ],
    )(c2d, t3d, p3d)


def _tc_body(c_ref, t_ref, p_ref, num_ref, den_ref, accn, accd):
    j = pl.program_id(0)

    @pl.when(j == 0)
    def _():
        accn[...] = jnp.zeros((TC_BB, TC_BN), jnp.float32)
        accd[...] = jnp.zeros((TC_BB, TC_BN), jnp.float32)

    col = lax.broadcasted_iota(jnp.int32, (TC_BB, TC_BN), 1)
    valid = ((j + TC_JOFF) * TC_BN + col) < N
    w = jnp.where(valid & (c_ref[...] != 0), jnp.float32(1), jnp.float32(0))
    d = p_ref[...] - t_ref[...]
    s = jnp.sum(d * d, axis=1)
    accn[...] += jnp.where(w != 0, s, jnp.float32(0))
    accd[...] += w

    @pl.when(j == TC_NBLK - 1)
    def _():
        num_ref[0, 0] = jnp.sum(accn[...])
        den_ref[0, 0] = jnp.sum(accd[...])


def _tc_call(c2d, t3d, p3d):
    return pl.pallas_call(
        _tc_body,
        grid=(TC_NBLK,),
        in_specs=[
            pl.BlockSpec((TC_BB, TC_BN), lambda j: (0, j + TC_JOFF)),
            pl.BlockSpec((TC_BB, 4, TC_BN), lambda j: (0, 0, j + TC_JOFF)),
            pl.BlockSpec((TC_BB, 4, TC_BN), lambda j: (0, 0, j + TC_JOFF)),
        ],
        out_specs=[
            pl.BlockSpec(memory_space=pltpu.SMEM),
            pl.BlockSpec(memory_space=pltpu.SMEM),
        ],
        out_shape=[
            jax.ShapeDtypeStruct((1, 1), jnp.float32),
            jax.ShapeDtypeStruct((1, 1), jnp.float32),
        ],
        scratch_shapes=[
            pltpu.VMEM((TC_BB, TC_BN), jnp.float32),
            pltpu.VMEM((TC_BB, TC_BN), jnp.float32),
        ],
    )(c2d, t3d, p3d)


@jax.jit
def _bbox_mse(c2d, t3d, p3d):
    partials = _sc_call(c2d, t3d, p3d)
    tc_num, tc_den = _tc_call(c2d, t3d, p3d)
    num = jnp.sum(partials[:, :16]) + tc_num[0, 0]
    den = (jnp.sum(partials[:, 16:]) + tc_den[0, 0]) * 4.0
    return num / den


def kernel(target_bbox, target_class, predicted_bbox):
    t3d = jnp.transpose(target_bbox, (0, 2, 1))
    p3d = jnp.transpose(predicted_bbox, (0, 2, 1))
    c2d = target_class.astype(jnp.int32)
    return _bbox_mse(c2d, t3d, p3d)


# speedup vs baseline: 1.0905x; 1.0064x over previous
import jax
import jax.numpy as jnp
from jax import lax
from jax.experimental import pallas as pl
from jax.experimental.pallas import tpu as pltpu
from jax.experimental.pallas import tpu_sc as plsc

B = 64
N = 20000

SC_TILES = 84
N_SC = SC_TILES * 128

NUM_WORKERS = 32
OCTET = 8
QUARTER = N_SC // 4
CHUNK_N = 896
NUM_CHUNKS = QUARTER // CHUNK_N

TC_BB = 64
TC_BN = 1536
TC_JOFF = N_SC // TC_BN
TC_NBLK = -(-(N - N_SC) // TC_BN)


def _sc_body(c_hbm, t_hbm, p_hbm, out_hbm,
             c0, c1, t0, t1, p0, p1, acc_v,
             sc0, sc1, st0, st1, sp0, sp1):
    nc = 2
    wid = lax.axis_index("s") * nc + lax.axis_index("c")
    o = wid // 4
    q = wid % 4
    b0 = pl.multiple_of(o * OCTET, 8)
    nbase = q * QUARTER
    zero = jnp.zeros((16,), jnp.float32)

    def start(k, cv, tv, pv, cs, ts, ps):
        n0 = pl.multiple_of(nbase + k * CHUNK_N, 128)
        pltpu.make_async_copy(
            c_hbm.at[0, pl.ds(b0, OCTET), pl.ds(n0, CHUNK_N)], cv, cs).start()
        pltpu.make_async_copy(
            t_hbm.at[pl.ds(b0, OCTET), :, pl.ds(n0, CHUNK_N)], tv, ts).start()
        pltpu.make_async_copy(
            p_hbm.at[pl.ds(b0, OCTET), :, pl.ds(n0, CHUNK_N)], pv, ps).start()

    def wait(k, cv, tv, pv, cs, ts, ps):
        n0 = pl.multiple_of(nbase + k * CHUNK_N, 128)
        pltpu.make_async_copy(
            c_hbm.at[0, pl.ds(b0, OCTET), pl.ds(n0, CHUNK_N)], cv, cs).wait()
        pltpu.make_async_copy(
            t_hbm.at[pl.ds(b0, OCTET), :, pl.ds(n0, CHUNK_N)], tv, ts).wait()
        pltpu.make_async_copy(
            p_hbm.at[pl.ds(b0, OCTET), :, pl.ds(n0, CHUNK_N)], pv, ps).wait()

    def compute(cv, tv, pv, carry):
        def group_body(g, gcarry):
            gsq, gwa = gcarry
            col = pl.ds(g * 16, 16)
            for bi in range(OCTET):
                c16 = cv[bi, col]
                w = jnp.where(c16 != 0, jnp.float32(1), jnp.float32(0))
                gwa = gwa + w
                d = pv[bi, 0, col] - tv[bi, 0, col]
                s = d * d
                for comp in range(1, 4):
                    d = pv[bi, comp, col] - tv[bi, comp, col]
                    s = s + d * d
                gsq = gsq + s * w
            return gsq, gwa

        return lax.fori_loop(0, CHUNK_N // 16, group_body, carry)

    start(0, c0, t0, p0, sc0, st0, sp0)

    def pair_body(i, carry):
        ka = 2 * i
        wait(ka, c0, t0, p0, sc0, st0, sp0)
        start(ka + 1, c1, t1, p1, sc1, st1, sp1)
        carry = compute(c0, t0, p0, carry)
        wait(ka + 1, c1, t1, p1, sc1, st1, sp1)
        start(ka + 2, c0, t0, p0, sc0, st0, sp0)
        return compute(c1, t1, p1, carry)

    carry = lax.fori_loop(0, NUM_CHUNKS // 2, pair_body, (zero, zero))
    wait(NUM_CHUNKS - 1, c0, t0, p0, sc0, st0, sp0)
    sq, wa = compute(c0, t0, p0, carry)

    acc_v[pl.ds(0, 16)] = sq
    acc_v[pl.ds(16, 16)] = wa
    pltpu.sync_copy(acc_v, out_hbm.at[wid])


def _sc_call(c2d, t3d, p3d):
    c3d = c2d.reshape(1, B, N)
    mesh = plsc.VectorSubcoreMesh(core_axis_name="c", subcore_axis_name="s")
    return pl.kernel(
        _sc_body,
        mesh=mesh,
        out_type=jax.ShapeDtypeStruct((NUM_WORKERS, 32), jnp.float32),
        scratch_types=[
            pltpu.VMEM((OCTET, CHUNK_N), jnp.int32),
            pltpu.VMEM((OCTET, CHUNK_N), jnp.int32),
            pltpu.VMEM((OCTET, 4, CHUNK_N), jnp.float32),
            pltpu.VMEM((OCTET, 4, CHUNK_N), jnp.float32),
            pltpu.VMEM((OCTET, 4, CHUNK_N), jnp.float32),
            pltpu.VMEM((OCTET, 4, CHUNK_N), jnp.float32),
            pltpu.VMEM((32,), jnp.float32),
            pltpu.SemaphoreType.DMA,
            pltpu.SemaphoreType.DMA,
            pltpu.SemaphoreType.DMA,
            pltpu.SemaphoreType.DMA,
            pltpu.SemaphoreType.DMA,
            pltpu.SemaphoreType.DMA,
        ],
    )(c3d, t3d, p3d)


def _tc_body(c_ref, t_ref, p_ref, num_ref, den_ref, accn, accd):
    j = pl.program_id(0)

    @pl.when(j == 0)
    def _():
        accn[...] = jnp.zeros((TC_BB, TC_BN), jnp.float32)
        accd[...] = jnp.zeros((TC_BB, TC_BN), jnp.float32)

    col = lax.broadcasted_iota(jnp.int32, (TC_BB, TC_BN), 1)
    valid = ((j + TC_JOFF) * TC_BN + col) < N
    w = jnp.where(valid & (c_ref[...] != 0), jnp.float32(1), jnp.float32(0))
    d = p_ref[...] - t_ref[...]
    s = jnp.sum(d * d, axis=1)
    accn[...] += jnp.where(w != 0, s, jnp.float32(0))
    accd[...] += w

    @pl.when(j == TC_NBLK - 1)
    def _():
        num_ref[0, 0] = jnp.sum(accn[...])
        den_ref[0, 0] = jnp.sum(accd[...])


def _tc_call(c2d, t3d, p3d):
    return pl.pallas_call(
        _tc_body,
        grid=(TC_NBLK,),
        in_specs=[
            pl.BlockSpec((TC_BB, TC_BN), lambda j: (0, j + TC_JOFF)),
            pl.BlockSpec((TC_BB, 4, TC_BN), lambda j: (0, 0, j + TC_JOFF)),
            pl.BlockSpec((TC_BB, 4, TC_BN), lambda j: (0, 0, j + TC_JOFF)),
        ],
        out_specs=[
            pl.BlockSpec(memory_space=pltpu.SMEM),
            pl.BlockSpec(memory_space=pltpu.SMEM),
        ],
        out_shape=[
            jax.ShapeDtypeStruct((1, 1), jnp.float32),
            jax.ShapeDtypeStruct((1, 1), jnp.float32),
        ],
        scratch_shapes=[
            pltpu.VMEM((TC_BB, TC_BN), jnp.float32),
            pltpu.VMEM((TC_BB, TC_BN), jnp.float32),
        ],
    )(c2d, t3d, p3d)


@jax.jit
def _bbox_mse(c2d, t3d, p3d):
    partials = _sc_call(c2d, t3d, p3d)
    tc_num, tc_den = _tc_call(c2d, t3d, p3d)
    num = jnp.sum(partials[:, :16]) + tc_num[0, 0]
    den = (jnp.sum(partials[:, 16:]) + tc_den[0, 0]) * 4.0
    return num / den


def kernel(target_bbox, target_class, predicted_bbox):
    t3d = jnp.transpose(target_bbox, (0, 2, 1))
    p3d = jnp.transpose(predicted_bbox, (0, 2, 1))
    c2d = target_class.astype(jnp.int32)
    return _bbox_mse(c2d, t3d, p3d)
